# Initial kernel scaffold; baseline (speedup 1.0000x reference)
import jax
import jax.numpy as jnp
from jax.experimental import pallas as pl


def kernel(x, edge_index, edge_attr, W1, b1, W2, b2, att_vec):
    raise NotImplementedError("write your pallas kernel here")


# speedup vs baseline: 5.1756x; 5.1756x over previous
import functools

import jax
import jax.numpy as jnp
from jax import lax
from jax.experimental import pallas as pl
from jax.experimental.pallas import tpu as pltpu
from jax.experimental.pallas import tpu_sc as plsc

D = 128
C = 80
TE = 2000


def _sc_mesh():
    return plsc.VectorSubcoreMesh(core_axis_name="c", subcore_axis_name="s")


def _make_gather(n_nodes, n_edges, nc, ns):
    nw = nc * ns
    per_w = n_edges // nw
    iters = per_w // C

    @functools.partial(
        pl.kernel,
        mesh=_sc_mesh(),
        compiler_params=pltpu.CompilerParams(needs_layout_passes=False),
        out_type=jax.ShapeDtypeStruct((n_edges, D), jnp.float32),
        scratch_types=[
            pltpu.VMEM((C,), jnp.int32),
            pltpu.VMEM((C, D), jnp.float32),
            pltpu.SemaphoreType.DMA,
        ],
    )
    def gather_k(x_hbm, src_hbm, g_hbm, idx_v, rows_v, sem):
        wid = lax.axis_index("s") * nc + lax.axis_index("c")

        def body(i, carry):
            base = wid * per_w + i * C
            pltpu.sync_copy(src_hbm.at[pl.ds(base, C)], idx_v)
            pltpu.async_copy(x_hbm.at[idx_v], rows_v, sem).wait()
            pltpu.sync_copy(rows_v, g_hbm.at[pl.ds(base, C)])
            return carry

        lax.fori_loop(0, iters, body, 0)

    return gather_k


def _make_scatter(n_nodes, n_edges, nc, ns):
    nw = nc * ns
    per_w = n_edges // nw
    iters = per_w // C
    n_chunks = n_nodes // 8
    chunks_per_tile = (n_chunks + ns - 1) // ns

    @functools.partial(
        pl.kernel,
        mesh=_sc_mesh(),
        compiler_params=pltpu.CompilerParams(needs_layout_passes=False),
        out_type=(
            jax.ShapeDtypeStruct((2, n_nodes, D), jnp.float32),
            jax.ShapeDtypeStruct((2, n_nodes), jnp.float32),
        ),
        scratch_types=[
            pltpu.VMEM((C,), jnp.int32),
            pltpu.VMEM((C, D), jnp.float32),
            pltpu.VMEM((C * 8,), jnp.float32),
            pltpu.VMEM((C,), jnp.float32),
            pltpu.VMEM_SHARED((n_nodes, D), jnp.float32),
            pltpu.VMEM_SHARED((n_nodes,), jnp.float32),
        ],
    )
    def scatter_k(p_hbm, tgt_hbm, e8_hbm, z2_hbm, z1_hbm,
                  part_hbm, den_hbm, idx_v, rows_v, e8_v, e_v, acc, dacc):
        cid = lax.axis_index("c")
        sid = lax.axis_index("s")
        wid = sid * nc + cid
        lanes = lax.iota(jnp.int32, 16)
        zero16 = lanes * 0

        def zero(k, carry):
            chunk = sid + k * ns

            @pl.when(chunk < n_chunks)
            def _():
                lo = chunk * 8
                pltpu.sync_copy(z2_hbm.at[pl.ds(lo, 8)], acc.at[pl.ds(lo, 8)])

            return carry

        lax.fori_loop(0, chunks_per_tile, zero, 0)

        @pl.when(sid == 0)
        def _():
            pltpu.sync_copy(z1_hbm, dacc)

        plsc.subcore_barrier()

        def body(i, carry):
            base = wid * per_w + i * C
            pltpu.sync_copy(tgt_hbm.at[pl.ds(base, C)], idx_v)
            pltpu.sync_copy(p_hbm.at[pl.ds(base, C)], rows_v)
            pltpu.sync_copy(e8_hbm.at[pl.ds(base * 8, C * 8)], e8_v)
            for j in range(C // 16):
                row = (lanes + (j * 16)) * 8
                e_v[pl.ds(j * 16, 16)] = plsc.load_gather(e8_v, [row])
            pltpu.sync_copy(rows_v, acc.at[idx_v], add=True)
            pltpu.sync_copy(e_v, dacc.at[idx_v], add=True)
            return carry

        lax.fori_loop(0, iters, body, 0)
        plsc.subcore_barrier()

        def wout(k, carry):
            chunk = sid + k * ns

            @pl.when(chunk < n_chunks)
            def _():
                lo = chunk * 8
                pltpu.sync_copy(acc.at[pl.ds(lo, 8)],
                                part_hbm.at[cid, pl.ds(lo, 8)])

            return carry

        lax.fori_loop(0, chunks_per_tile, wout, 0)

        @pl.when(sid == 0)
        def _():
            pltpu.sync_copy(dacc, den_hbm.at[cid])

    return scatter_k


def _make_finalize(n_nodes, n_edges, nc, ns):
    nw = nc * ns
    per_w = n_edges // nw
    iters = per_w // C
    n_chunks = n_nodes // 8
    chunks_per_w = (n_chunks + nw - 1) // nw

    @functools.partial(
        pl.kernel,
        mesh=_sc_mesh(),
        compiler_params=pltpu.CompilerParams(needs_layout_passes=False),
        out_type=(
            jax.ShapeDtypeStruct((n_nodes, D), jnp.float32),
            jax.ShapeDtypeStruct((n_edges,), jnp.float32),
        ),
        scratch_types=[
            pltpu.VMEM((n_nodes,), jnp.float32),
            pltpu.VMEM((n_nodes,), jnp.float32),
            pltpu.VMEM((8, D), jnp.float32),
            pltpu.VMEM((8, D), jnp.float32),
            pltpu.VMEM((C * 8,), jnp.float32),
            pltpu.VMEM((C,), jnp.int32),
            pltpu.VMEM((C,), jnp.float32),
        ],
    )
    def fin_k(part_hbm, denp_hbm, e8_hbm, tgt_hbm, out_hbm, att_hbm,
              den_v, d1_v, r0_v, r1_v, e8_v, t_v, a_v):
        wid = lax.axis_index("s") * nc + lax.axis_index("c")
        lanes = lax.iota(jnp.int32, 16)
        zero16 = lanes * 0

        pltpu.sync_copy(denp_hbm.at[0], den_v)
        pltpu.sync_copy(denp_hbm.at[1], d1_v)

        def dsum(i, carry):
            sl = pl.ds(i * 16, 16)
            den_v[sl] = den_v[sl] + d1_v[sl]
            return carry

        lax.fori_loop(0, n_nodes // 16, dsum, 0)

        def rows(k, carry):
            chunk = wid + k * nw

            @pl.when(chunk < n_chunks)
            def _():
                lo = chunk * 8
                pltpu.sync_copy(part_hbm.at[0, pl.ds(lo, 8)], r0_v)
                pltpu.sync_copy(part_hbm.at[1, pl.ds(lo, 8)], r1_v)
                for r in range(8):
                    dv = plsc.load_gather(den_v, [zero16 + (lo + r)])
                    ok = dv > 0.0
                    dsafe = jnp.where(ok, dv, 1.0)
                    for cc in range(D // 16):
                        sl = pl.ds(cc * 16, 16)
                        q = (r0_v[r, sl] + r1_v[r, sl]) / dsafe
                        r0_v[r, sl] = jnp.where(ok, q, 0.0)
                pltpu.sync_copy(r0_v, out_hbm.at[pl.ds(lo, 8)])

            return carry

        lax.fori_loop(0, chunks_per_w, rows, 0)

        def att(i, carry):
            base = wid * per_w + i * C
            pltpu.sync_copy(e8_hbm.at[pl.ds(base * 8, C * 8)], e8_v)
            pltpu.sync_copy(tgt_hbm.at[pl.ds(base, C)], t_v)
            for j in range(C // 16):
                row = (lanes + (j * 16)) * 8
                sl = pl.ds(j * 16, 16)
                ev = plsc.load_gather(e8_v, [row])
                dv = plsc.load_gather(den_v, [t_v[sl]])
                a_v[sl] = ev / dv
            pltpu.sync_copy(a_v, att_hbm.at[pl.ds(base, C)])
            return carry

        lax.fori_loop(0, iters, att, 0)

    return fin_k


def _tc_edge_math(g, edge_attr, w1t, b1, w2t, b2, att):
    n_edges = g.shape[0]

    def body(g_ref, ea_ref, w1t_ref, b1_ref, w2t_ref, b2_ref, att_ref,
             p_ref, e8_ref):
        h = jnp.tanh(
            jnp.dot(ea_ref[...], w1t_ref[...],
                    preferred_element_type=jnp.float32) + b1_ref[...])
        w = jnp.dot(h, w2t_ref[...],
                    preferred_element_type=jnp.float32) + b2_ref[...]
        m = g_ref[...] * w
        s = jnp.sum(m * att_ref[...], axis=1, keepdims=True)
        e = jnp.exp(s)
        p_ref[...] = m * e
        e8_ref[...] = e * jnp.ones((1, 8), jnp.float32)

    grid = (n_edges // TE,)
    return pl.pallas_call(
        body,
        grid=grid,
        in_specs=[
            pl.BlockSpec((TE, D), lambda i: (i, 0)),
            pl.BlockSpec((TE, 16), lambda i: (i, 0)),
            pl.BlockSpec((16, 128), lambda i: (0, 0)),
            pl.BlockSpec((1, 128), lambda i: (0, 0)),
            pl.BlockSpec((128, 128), lambda i: (0, 0)),
            pl.BlockSpec((1, 128), lambda i: (0, 0)),
            pl.BlockSpec((1, 128), lambda i: (0, 0)),
        ],
        out_specs=[
            pl.BlockSpec((TE, D), lambda i: (i, 0)),
            pl.BlockSpec((TE, 8), lambda i: (i, 0)),
        ],
        out_shape=[
            jax.ShapeDtypeStruct((n_edges, D), jnp.float32),
            jax.ShapeDtypeStruct((n_edges, 8), jnp.float32),
        ],
    )(g, edge_attr, w1t, b1, w2t, b2, att)


def kernel(x, edge_index, edge_attr, W1, b1, W2, b2, att_vec):
    n_nodes = x.shape[0]
    n_edges = edge_attr.shape[0]
    info = plsc.get_sparse_core_info()
    nc, ns = info.num_cores, info.num_subcores

    src = edge_index[0].astype(jnp.int32)
    tgt = edge_index[1].astype(jnp.int32)

    g = _make_gather(n_nodes, n_edges, nc, ns)(x, src)

    p, e8 = _tc_edge_math(
        g, edge_attr, W1.T, b1.reshape(1, 128), W2.T, b2.reshape(1, 128),
        att_vec.reshape(1, 128))

    e8f = e8.reshape(-1)
    z2 = jnp.zeros((n_nodes, D), jnp.float32)
    z1 = jnp.zeros((n_nodes,), jnp.float32)
    part, den_part = _make_scatter(n_nodes, n_edges, nc, ns)(
        p, tgt, e8f, z2, z1)

    out, att = _make_finalize(n_nodes, n_edges, nc, ns)(
        part, den_part, e8f, tgt)

    return (out, att)


# speedup vs baseline: 6.9727x; 1.3472x over previous
import functools

import jax
import jax.numpy as jnp
from jax import lax
from jax.experimental import pallas as pl
from jax.experimental.pallas import tpu as pltpu
from jax.experimental.pallas import tpu_sc as plsc

D = 128
C = 80
TE = 512
NR = 40


def _sc_mesh():
    return plsc.VectorSubcoreMesh(core_axis_name="c", subcore_axis_name="s")


def _make_gather(n_nodes, n_edges, nc, ns):
    nw = nc * ns
    per_w = n_edges // nw
    iters = per_w // C

    @functools.partial(
        pl.kernel,
        mesh=_sc_mesh(),
        compiler_params=pltpu.CompilerParams(needs_layout_passes=False),
        out_type=jax.ShapeDtypeStruct((n_edges, D), jnp.float32),
        scratch_types=[
            pltpu.VMEM((per_w,), jnp.int32),
            pltpu.VMEM((C, D), jnp.float32),
            pltpu.VMEM((C, D), jnp.float32),
            pltpu.SemaphoreType.DMA,
            pltpu.SemaphoreType.DMA,
        ],
    )
    def gather_k(x_hbm, src_hbm, g_hbm, idx_all, rows0, rows1, g0, g1):
        wid = lax.axis_index("s") * nc + lax.axis_index("c")
        pltpu.sync_copy(src_hbm.at[pl.ds(wid * per_w, per_w)], idx_all)

        def start(c, rows_b, sem_b):
            pltpu.async_copy(
                x_hbm.at[idx_all.at[pl.ds(c * C, C)]], rows_b, sem_b)

        def wait(c, rows_b, sem_b):
            pltpu.make_async_copy(
                x_hbm.at[idx_all.at[pl.ds(c * C, C)]], rows_b, sem_b).wait()

        def write(c, rows_b):
            pltpu.sync_copy(rows_b, g_hbm.at[pl.ds(wid * per_w + c * C, C)])

        start(0, rows0, g0)

        def body(k, carry):
            c0 = 2 * k
            c1 = 2 * k + 1
            start(c1, rows1, g1)
            wait(c0, rows0, g0)
            write(c0, rows0)
            start(c0 + 2, rows0, g0)
            wait(c1, rows1, g1)
            write(c1, rows1)
            return carry

        lax.fori_loop(0, (iters - 1) // 2, body, 0)
        wait(iters - 1, rows0, g0)
        write(iters - 1, rows0)

    return gather_k


def _make_scatter(n_nodes, n_edges, nc, ns):
    nw = nc * ns
    per_w = n_edges // nw
    iters = per_w // C
    n_chunks = n_nodes // NR
    chunks_per_tile = (n_chunks + ns - 1) // ns

    @functools.partial(
        pl.kernel,
        mesh=_sc_mesh(),
        compiler_params=pltpu.CompilerParams(needs_layout_passes=False),
        out_type=(
            jax.ShapeDtypeStruct((2, n_nodes, D), jnp.float32),
            jax.ShapeDtypeStruct((2, n_nodes), jnp.float32),
        ),
        scratch_types=[
            pltpu.VMEM((iters, C), jnp.int32),
            pltpu.VMEM((per_w,), jnp.float32),
            pltpu.VMEM((C, D), jnp.float32),
            pltpu.VMEM((C, D), jnp.float32),
            pltpu.VMEM_SHARED((n_nodes, D), jnp.float32),
            pltpu.VMEM_SHARED((n_nodes,), jnp.float32),
            pltpu.SemaphoreType.DMA,
            pltpu.SemaphoreType.DMA,
        ],
    )
    def scatter_k(p_hbm, tgt3_hbm, e_hbm, z2_hbm, z1_hbm,
                  part_hbm, den_hbm, idx2, e_all, rows0, rows1, acc, dacc,
                  l0, l1):
        cid = lax.axis_index("c")
        sid = lax.axis_index("s")
        wid = sid * nc + cid

        pltpu.sync_copy(tgt3_hbm.at[wid], idx2)
        pltpu.sync_copy(e_hbm.at[pl.ds(wid * per_w, per_w)], e_all)

        def zero(k, carry):
            chunk = sid + k * ns

            @pl.when(chunk < n_chunks)
            def _():
                lo = chunk * NR
                pltpu.sync_copy(z2_hbm.at[pl.ds(lo, NR)],
                                acc.at[pl.ds(lo, NR)])

            return carry

        lax.fori_loop(0, chunks_per_tile, zero, 0)

        @pl.when(sid == 0)
        def _():
            pltpu.sync_copy(z1_hbm, dacc)

        plsc.subcore_barrier()

        def start(c, rows_b, sem_b):
            pltpu.async_copy(
                p_hbm.at[pl.ds(wid * per_w + c * C, C)], rows_b, sem_b)

        def wait(c, rows_b, sem_b):
            pltpu.make_async_copy(
                p_hbm.at[pl.ds(wid * per_w + c * C, C)], rows_b, sem_b).wait()

        def scat(c, rows_b):
            pltpu.sync_copy(rows_b, acc.at[idx2.at[c]], add=True)
            pltpu.sync_copy(e_all.at[pl.ds(c * C, C)], dacc.at[idx2.at[c]],
                            add=True)

        start(0, rows0, l0)

        def body(k, carry):
            c0 = 2 * k
            c1 = 2 * k + 1
            start(c1, rows1, l1)
            wait(c0, rows0, l0)
            scat(c0, rows0)
            start(c0 + 2, rows0, l0)
            wait(c1, rows1, l1)
            scat(c1, rows1)
            return carry

        lax.fori_loop(0, (iters - 1) // 2, body, 0)
        wait(iters - 1, rows0, l0)
        scat(iters - 1, rows0)

        plsc.subcore_barrier()

        def wout(k, carry):
            chunk = sid + k * ns

            @pl.when(chunk < n_chunks)
            def _():
                lo = chunk * NR
                pltpu.sync_copy(acc.at[pl.ds(lo, NR)],
                                part_hbm.at[cid, pl.ds(lo, NR)])

            return carry

        lax.fori_loop(0, chunks_per_tile, wout, 0)

        @pl.when(sid == 0)
        def _():
            pltpu.sync_copy(dacc, den_hbm.at[cid])

    return scatter_k


def _make_finalize(n_nodes, n_edges, nc, ns):
    nw = nc * ns
    per_w = n_edges // nw
    iters = per_w // C
    n_chunks = n_nodes // NR
    chunks_per_w = (n_chunks + nw - 1) // nw

    @functools.partial(
        pl.kernel,
        mesh=_sc_mesh(),
        compiler_params=pltpu.CompilerParams(needs_layout_passes=False),
        out_type=(
            jax.ShapeDtypeStruct((n_nodes, D), jnp.float32),
            jax.ShapeDtypeStruct((n_edges,), jnp.float32),
        ),
        scratch_types=[
            pltpu.VMEM((n_nodes,), jnp.float32),
            pltpu.VMEM((n_nodes,), jnp.float32),
            pltpu.VMEM((NR, D), jnp.float32),
            pltpu.VMEM((NR, D), jnp.float32),
            pltpu.VMEM((per_w,), jnp.float32),
            pltpu.VMEM((per_w,), jnp.int32),
            pltpu.VMEM((per_w,), jnp.float32),
        ],
    )
    def fin_k(part_hbm, denp_hbm, e_hbm, tgt_hbm, out_hbm, att_hbm,
              den_v, d1_v, r0_v, r1_v, e_all, t_all, a_all):
        wid = lax.axis_index("s") * nc + lax.axis_index("c")
        lanes = lax.iota(jnp.int32, 16)
        zero16 = lanes * 0

        pltpu.sync_copy(denp_hbm.at[0], den_v)
        pltpu.sync_copy(denp_hbm.at[1], d1_v)
        pltpu.sync_copy(e_hbm.at[pl.ds(wid * per_w, per_w)], e_all)
        pltpu.sync_copy(tgt_hbm.at[pl.ds(wid * per_w, per_w)], t_all)

        def dsum(i, carry):
            sl = pl.ds(i * 16, 16)
            den_v[sl] = den_v[sl] + d1_v[sl]
            return carry

        lax.fori_loop(0, n_nodes // 16, dsum, 0)

        def rows(k, carry):
            chunk = wid + k * nw

            @pl.when(chunk < n_chunks)
            def _():
                lo = chunk * NR
                pltpu.sync_copy(part_hbm.at[0, pl.ds(lo, NR)], r0_v)
                pltpu.sync_copy(part_hbm.at[1, pl.ds(lo, NR)], r1_v)
                for r in range(NR):
                    dv = plsc.load_gather(den_v, [zero16 + (lo + r)])
                    ok = dv > 0.0
                    dsafe = jnp.where(ok, dv, 1.0)
                    for cc in range(D // 16):
                        sl = pl.ds(cc * 16, 16)
                        q = (r0_v[r, sl] + r1_v[r, sl]) / dsafe
                        r0_v[r, sl] = jnp.where(ok, q, 0.0)
                pltpu.sync_copy(r0_v, out_hbm.at[pl.ds(lo, NR)])

            return carry

        lax.fori_loop(0, chunks_per_w, rows, 0)

        def att(i, carry):
            sl = pl.ds(i * 16, 16)
            dv = plsc.load_gather(den_v, [t_all[sl]])
            a_all[sl] = e_all[sl] / dv
            return carry

        lax.fori_loop(0, per_w // 16, att, 0)
        pltpu.sync_copy(a_all, att_hbm.at[pl.ds(wid * per_w, per_w)])

    return fin_k


def _tc_edge_math(g, edge_attr, w1t, b1, w2t, b2, att):
    n_edges = g.shape[0]

    def body(g_ref, ea_ref, w1t_ref, b1_ref, w2t_ref, b2_ref, att_ref,
             p_ref, e_ref):
        h = jnp.tanh(
            jnp.dot(ea_ref[...], w1t_ref[...],
                    preferred_element_type=jnp.float32) + b1_ref[...])
        w = jnp.dot(h, w2t_ref[...],
                    preferred_element_type=jnp.float32) + b2_ref[...]
        m = g_ref[...] * w
        s = jnp.sum(m * att_ref[...], axis=1, keepdims=True)
        e = jnp.exp(s)
        p_ref[...] = m * e
        e_ref[...] = e[:, 0]

    grid = (n_edges // TE,)
    return pl.pallas_call(
        body,
        grid=grid,
        in_specs=[
            pl.BlockSpec((TE, D), lambda i: (i, 0)),
            pl.BlockSpec((TE, 16), lambda i: (i, 0)),
            pl.BlockSpec((16, 128), lambda i: (0, 0)),
            pl.BlockSpec((1, 128), lambda i: (0, 0)),
            pl.BlockSpec((128, 128), lambda i: (0, 0)),
            pl.BlockSpec((1, 128), lambda i: (0, 0)),
            pl.BlockSpec((1, 128), lambda i: (0, 0)),
        ],
        out_specs=[
            pl.BlockSpec((TE, D), lambda i: (i, 0)),
            pl.BlockSpec((TE,), lambda i: (i,)),
        ],
        out_shape=[
            jax.ShapeDtypeStruct((n_edges, D), jnp.float32),
            jax.ShapeDtypeStruct((n_edges,), jnp.float32),
        ],
    )(g, edge_attr, w1t, b1, w2t, b2, att)


def kernel(x, edge_index, edge_attr, W1, b1, W2, b2, att_vec):
    n_nodes = x.shape[0]
    n_edges = edge_attr.shape[0]
    info = plsc.get_sparse_core_info()
    nc, ns = info.num_cores, info.num_subcores
    nw = nc * ns
    per_w = n_edges // nw

    src = edge_index[0].astype(jnp.int32)
    tgt = edge_index[1].astype(jnp.int32)
    tgt3 = tgt.reshape(nw, per_w // C, C)

    g = _make_gather(n_nodes, n_edges, nc, ns)(x, src)

    p, e = _tc_edge_math(
        g, edge_attr, W1.T, b1.reshape(1, 128), W2.T, b2.reshape(1, 128),
        att_vec.reshape(1, 128))

    z2 = jnp.zeros((n_nodes, D), jnp.float32)
    z1 = jnp.zeros((n_nodes,), jnp.float32)
    part, den_part = _make_scatter(n_nodes, n_edges, nc, ns)(
        p, tgt3, e, z2, z1)

    out, att = _make_finalize(n_nodes, n_edges, nc, ns)(
        part, den_part, e, tgt)

    return (out, att)


# speedup vs baseline: 10.3695x; 1.4872x over previous
import functools

import jax
import jax.numpy as jnp
from jax import lax
from jax.experimental import pallas as pl
from jax.experimental.pallas import tpu as pltpu
from jax.experimental.pallas import tpu_sc as plsc

D = 128
C = 80
TE = 6400
NR = 40


def _sc_mesh():
    return plsc.VectorSubcoreMesh(core_axis_name="c", subcore_axis_name="s")


def _make_gather(n_nodes, n_edges, nc, ns):
    nw = nc * ns
    per_w = n_edges // nw
    iters = per_w // C
    n_chunks = n_nodes // NR
    chunks_per_tile = (n_chunks + ns - 1) // ns

    @functools.partial(
        pl.kernel,
        mesh=_sc_mesh(),
        compiler_params=pltpu.CompilerParams(needs_layout_passes=False),
        out_type=jax.ShapeDtypeStruct((n_edges, D), jnp.float32),
        scratch_types=[
            pltpu.VMEM((per_w,), jnp.int32),
            pltpu.VMEM((C, D), jnp.float32),
            pltpu.VMEM((C, D), jnp.float32),
            pltpu.VMEM_SHARED((n_nodes, D), jnp.float32),
            pltpu.SemaphoreType.DMA,
            pltpu.SemaphoreType.DMA,
        ],
    )
    def gather_k(x_hbm, src_hbm, g_hbm, idx_all, rows0, rows1, xs, g0, g1):
        cid = lax.axis_index("c")
        sid = lax.axis_index("s")
        wid = sid * nc + cid
        pltpu.sync_copy(src_hbm.at[pl.ds(wid * per_w, per_w)], idx_all)

        def stage(k, carry):
            chunk = sid + k * ns

            @pl.when(chunk < n_chunks)
            def _():
                lo = chunk * NR
                pltpu.sync_copy(x_hbm.at[pl.ds(lo, NR)], xs.at[pl.ds(lo, NR)])

            return carry

        lax.fori_loop(0, chunks_per_tile, stage, 0)
        plsc.subcore_barrier()

        def start(c, rows_b, sem_b):
            pltpu.async_copy(
                xs.at[idx_all.at[pl.ds(c * C, C)]], rows_b, sem_b)

        def wait(c, rows_b, sem_b):
            pltpu.make_async_copy(
                xs.at[idx_all.at[pl.ds(c * C, C)]], rows_b, sem_b).wait()

        def write(c, rows_b):
            pltpu.sync_copy(rows_b, g_hbm.at[pl.ds(wid * per_w + c * C, C)])

        start(0, rows0, g0)

        def body(k, carry):
            c0 = 2 * k
            c1 = 2 * k + 1
            start(c1, rows1, g1)
            wait(c0, rows0, g0)
            write(c0, rows0)
            start(c0 + 2, rows0, g0)
            wait(c1, rows1, g1)
            write(c1, rows1)
            return carry

        lax.fori_loop(0, (iters - 1) // 2, body, 0)
        wait(iters - 1, rows0, g0)
        write(iters - 1, rows0)

    return gather_k


def _make_scatter(n_nodes, n_edges, nc, ns):
    nw = nc * ns
    per_w = n_edges // nw
    iters = per_w // C
    n_chunks = n_nodes // NR
    chunks_per_tile = (n_chunks + ns - 1) // ns

    @functools.partial(
        pl.kernel,
        mesh=_sc_mesh(),
        compiler_params=pltpu.CompilerParams(needs_layout_passes=False),
        out_type=(
            jax.ShapeDtypeStruct((2, n_nodes, D), jnp.float32),
            jax.ShapeDtypeStruct((2, n_nodes), jnp.float32),
        ),
        scratch_types=[
            pltpu.VMEM((iters, C), jnp.int32),
            pltpu.VMEM((C * 8,), jnp.float32),
            pltpu.VMEM((C * 8,), jnp.float32),
            pltpu.VMEM((C,), jnp.float32),
            pltpu.VMEM((C,), jnp.float32),
            pltpu.VMEM((C, D), jnp.float32),
            pltpu.VMEM((C, D), jnp.float32),
            pltpu.VMEM_SHARED((n_nodes, D), jnp.float32),
            pltpu.VMEM_SHARED((n_nodes,), jnp.float32),
            pltpu.SemaphoreType.DMA,
            pltpu.SemaphoreType.DMA,
        ],
    )
    def scatter_k(p_hbm, tgt3_hbm, e_hbm, z2_hbm, z1_hbm,
                  part_hbm, den_hbm, idx2, e8c0, e8c1, eb0, eb1,
                  rows0, rows1, acc, dacc, l0, l1):
        cid = lax.axis_index("c")
        sid = lax.axis_index("s")
        wid = sid * nc + cid
        lanes = lax.iota(jnp.int32, 16)

        pltpu.sync_copy(tgt3_hbm.at[wid], idx2)

        def zero(k, carry):
            chunk = sid + k * ns

            @pl.when(chunk < n_chunks)
            def _():
                lo = chunk * NR
                pltpu.sync_copy(z2_hbm.at[pl.ds(lo, NR)],
                                acc.at[pl.ds(lo, NR)])

            return carry

        lax.fori_loop(0, chunks_per_tile, zero, 0)

        @pl.when(sid == 0)
        def _():
            pltpu.sync_copy(z1_hbm, dacc)

        plsc.subcore_barrier()

        def start(c, rows_b, e8c_b, sem_b):
            pltpu.async_copy(
                p_hbm.at[pl.ds(wid * per_w + c * C, C)], rows_b, sem_b)
            pltpu.async_copy(
                e_hbm.at[pl.ds((wid * per_w + c * C) * 8, C * 8)], e8c_b,
                sem_b)

        def wait(c, rows_b, e8c_b, sem_b):
            pltpu.make_async_copy(
                p_hbm.at[pl.ds(wid * per_w + c * C, C)], rows_b, sem_b).wait()
            pltpu.make_async_copy(
                e_hbm.at[pl.ds((wid * per_w + c * C) * 8, C * 8)], e8c_b,
                sem_b).wait()

        def scat(c, rows_b, e8c_b, e_b):
            for j in range(C // 16):
                row = (lanes + (j * 16)) * 8
                e_b[pl.ds(j * 16, 16)] = plsc.load_gather(e8c_b, [row])
            pltpu.sync_copy(rows_b, acc.at[idx2.at[c]], add=True)
            pltpu.sync_copy(e_b, dacc.at[idx2.at[c]], add=True)

        start(0, rows0, e8c0, l0)

        def body(k, carry):
            c0 = 2 * k
            c1 = 2 * k + 1
            start(c1, rows1, e8c1, l1)
            wait(c0, rows0, e8c0, l0)
            scat(c0, rows0, e8c0, eb0)
            start(c0 + 2, rows0, e8c0, l0)
            wait(c1, rows1, e8c1, l1)
            scat(c1, rows1, e8c1, eb1)
            return carry

        lax.fori_loop(0, (iters - 1) // 2, body, 0)
        wait(iters - 1, rows0, e8c0, l0)
        scat(iters - 1, rows0, e8c0, eb0)

        plsc.subcore_barrier()

        def wout(k, carry):
            chunk = sid + k * ns

            @pl.when(chunk < n_chunks)
            def _():
                lo = chunk * NR
                pltpu.sync_copy(acc.at[pl.ds(lo, NR)],
                                part_hbm.at[cid, pl.ds(lo, NR)])

            return carry

        lax.fori_loop(0, chunks_per_tile, wout, 0)

        @pl.when(sid == 0)
        def _():
            pltpu.sync_copy(dacc, den_hbm.at[cid])

    return scatter_k


def _make_finalize(n_nodes, n_edges, nc, ns):
    nw = nc * ns
    per_w = n_edges // nw
    iters = per_w // C
    n_chunks = n_nodes // NR
    chunks_per_w = (n_chunks + nw - 1) // nw

    @functools.partial(
        pl.kernel,
        mesh=_sc_mesh(),
        compiler_params=pltpu.CompilerParams(needs_layout_passes=False),
        out_type=(
            jax.ShapeDtypeStruct((n_nodes, D), jnp.float32),
            jax.ShapeDtypeStruct((n_edges,), jnp.float32),
        ),
        scratch_types=[
            pltpu.VMEM((n_nodes,), jnp.float32),
            pltpu.VMEM((NR, D), jnp.float32),
            pltpu.VMEM((NR, D), jnp.float32),
            pltpu.VMEM((per_w * 8,), jnp.float32),
            pltpu.VMEM((per_w,), jnp.int32),
            pltpu.VMEM((per_w,), jnp.float32),
        ],
    )
    def fin_k(part_hbm, denp_hbm, e_hbm, tgt_hbm, out_hbm, att_hbm,
              den_v, r0_v, r1_v, e8_all, t_all, a_all):
        wid = lax.axis_index("s") * nc + lax.axis_index("c")
        lanes = lax.iota(jnp.int32, 16)
        zero16 = lanes * 0

        pltpu.sync_copy(denp_hbm.at[0], den_v)
        pltpu.sync_copy(denp_hbm.at[1], a_all)
        pltpu.sync_copy(e_hbm.at[pl.ds(wid * per_w * 8, per_w * 8)], e8_all)
        pltpu.sync_copy(tgt_hbm.at[pl.ds(wid * per_w, per_w)], t_all)

        def dsum(i, carry):
            sl = pl.ds(i * 16, 16)
            den_v[sl] = den_v[sl] + a_all[sl]
            return carry

        lax.fori_loop(0, n_nodes // 16, dsum, 0)

        def rows(k, carry):
            chunk = wid + k * nw

            @pl.when(chunk < n_chunks)
            def _():
                lo = chunk * NR
                pltpu.sync_copy(part_hbm.at[0, pl.ds(lo, NR)], r0_v)
                pltpu.sync_copy(part_hbm.at[1, pl.ds(lo, NR)], r1_v)
                for r in range(NR):
                    dv = plsc.load_gather(den_v, [zero16 + (lo + r)])
                    ok = dv > 0.0
                    dsafe = jnp.where(ok, dv, 1.0)
                    for cc in range(D // 16):
                        sl = pl.ds(cc * 16, 16)
                        q = (r0_v[r, sl] + r1_v[r, sl]) / dsafe
                        r0_v[r, sl] = jnp.where(ok, q, 0.0)
                pltpu.sync_copy(r0_v, out_hbm.at[pl.ds(lo, NR)])

            return carry

        lax.fori_loop(0, chunks_per_w, rows, 0)

        def att(i, carry):
            sl = pl.ds(i * 16, 16)
            dv = plsc.load_gather(den_v, [t_all[sl]])
            ev = plsc.load_gather(e8_all, [(lanes + i * 16) * 8])
            a_all[sl] = ev / dv
            return carry

        lax.fori_loop(0, per_w // 16, att, 0)
        pltpu.sync_copy(a_all, att_hbm.at[pl.ds(wid * per_w, per_w)])

    return fin_k


def _tc_edge_math(g, edge_attr, w1t, b1, w2t, b2, att):
    n_edges = g.shape[0]

    def body(g_ref, ea_ref, w1t_ref, b1_ref, w2t_ref, b2_ref, att_ref,
             p_ref, e_ref):
        h = jnp.tanh(
            jnp.dot(ea_ref[...], w1t_ref[...],
                    preferred_element_type=jnp.float32) + b1_ref[...])
        w = jnp.dot(h, w2t_ref[...],
                    preferred_element_type=jnp.float32) + b2_ref[...]
        m = g_ref[...] * w
        e_full = jnp.exp(jnp.dot(m, att_ref[...],
                                 preferred_element_type=jnp.float32))
        p_ref[...] = m * e_full
        e_ref[...] = e_full[:, 0:8]

    grid = (n_edges // TE,)
    return pl.pallas_call(
        body,
        grid=grid,
        in_specs=[
            pl.BlockSpec((TE, D), lambda i: (i, 0)),
            pl.BlockSpec((TE, 16), lambda i: (i, 0)),
            pl.BlockSpec((16, 128), lambda i: (0, 0)),
            pl.BlockSpec((1, 128), lambda i: (0, 0)),
            pl.BlockSpec((128, 128), lambda i: (0, 0)),
            pl.BlockSpec((1, 128), lambda i: (0, 0)),
            pl.BlockSpec((128, 128), lambda i: (0, 0)),
        ],
        out_specs=[
            pl.BlockSpec((TE, D), lambda i: (i, 0)),
            pl.BlockSpec((TE, 8), lambda i: (i, 0)),
        ],
        out_shape=[
            jax.ShapeDtypeStruct((n_edges, D), jnp.float32),
            jax.ShapeDtypeStruct((n_edges, 8), jnp.float32),
        ],
    )(g, edge_attr, w1t, b1, w2t, b2, att)


def kernel(x, edge_index, edge_attr, W1, b1, W2, b2, att_vec):
    n_nodes = x.shape[0]
    n_edges = edge_attr.shape[0]
    info = plsc.get_sparse_core_info()
    nc, ns = info.num_cores, info.num_subcores
    nw = nc * ns
    per_w = n_edges // nw

    src = edge_index[0].astype(jnp.int32)
    tgt = edge_index[1].astype(jnp.int32)
    tgt3 = tgt.reshape(nw, per_w // C, C)

    g = _make_gather(n_nodes, n_edges, nc, ns)(x, src)

    att_mat = jnp.tile(att_vec, (1, 128))
    p, e8 = _tc_edge_math(
        g, edge_attr, W1.T, b1.reshape(1, 128), W2.T, b2.reshape(1, 128),
        att_mat)

    e8f = e8.reshape(-1)
    z2 = jnp.zeros((n_nodes, D), jnp.float32)
    z1 = jnp.zeros((n_nodes,), jnp.float32)
    part, den_part = _make_scatter(n_nodes, n_edges, nc, ns)(
        p, tgt3, e8f, z2, z1)

    out, att = _make_finalize(n_nodes, n_edges, nc, ns)(
        part, den_part, e8f, tgt)

    return (out, att)


# speedup vs baseline: 10.3742x; 1.0005x over previous
import functools

import jax
import jax.numpy as jnp
from jax import lax
from jax.experimental import pallas as pl
from jax.experimental.pallas import tpu as pltpu
from jax.experimental.pallas import tpu_sc as plsc

D = 128
C = 80
TE = 6400
NR = 40


def _sc_mesh():
    return plsc.VectorSubcoreMesh(core_axis_name="c", subcore_axis_name="s")


def _make_gather(n_nodes, n_edges, nc, ns):
    nw = nc * ns
    per_w = n_edges // nw
    iters = per_w // C
    n_chunks = n_nodes // NR
    chunks_per_tile = (n_chunks + ns - 1) // ns
    GR = 3
    ng = iters // GR
    tail = iters - ng * GR

    @functools.partial(
        pl.kernel,
        mesh=_sc_mesh(),
        compiler_params=pltpu.CompilerParams(needs_layout_passes=False),
        out_type=jax.ShapeDtypeStruct((n_edges, D), jnp.float32),
        scratch_types=[
            pltpu.VMEM((per_w,), jnp.int32),
            pltpu.VMEM((C, D), jnp.float32),
            pltpu.VMEM((C, D), jnp.float32),
            pltpu.VMEM((C, D), jnp.float32),
            pltpu.VMEM_SHARED((n_nodes, D), jnp.float32),
            pltpu.SemaphoreType.DMA,
            pltpu.SemaphoreType.DMA,
            pltpu.SemaphoreType.DMA,
            pltpu.SemaphoreType.DMA,
            pltpu.SemaphoreType.DMA,
            pltpu.SemaphoreType.DMA,
        ],
    )
    def gather_k(x_hbm, src_hbm, g_hbm, idx_all, r0, r1, r2, xs,
                 gs0, gs1, gs2, ws0, ws1, ws2):
        rows = [r0, r1, r2]
        gsem = [gs0, gs1, gs2]
        wsem = [ws0, ws1, ws2]
        cid = lax.axis_index("c")
        sid = lax.axis_index("s")
        wid = sid * nc + cid
        pltpu.sync_copy(src_hbm.at[pl.ds(wid * per_w, per_w)], idx_all)

        def stage(k, carry):
            chunk = sid + k * ns

            @pl.when(chunk < n_chunks)
            def _():
                lo = chunk * NR
                pltpu.sync_copy(x_hbm.at[pl.ds(lo, NR)], xs.at[pl.ds(lo, NR)])

            return carry

        lax.fori_loop(0, chunks_per_tile, stage, 0)
        plsc.subcore_barrier()

        def g_start(c, b):
            pltpu.async_copy(
                xs.at[idx_all.at[pl.ds(c * C, C)]], rows[b], gsem[b])

        def g_wait(c, b):
            pltpu.make_async_copy(
                xs.at[idx_all.at[pl.ds(c * C, C)]], rows[b], gsem[b]).wait()

        def w_start(c, b):
            pltpu.async_copy(
                rows[b], g_hbm.at[pl.ds(wid * per_w + c * C, C)], wsem[b])

        def w_wait(c, b):
            pltpu.make_async_copy(
                rows[b], g_hbm.at[pl.ds(wid * per_w + c * C, C)],
                wsem[b]).wait()

        def body(k, carry):
            for b in range(GR):
                c = k * GR + b

                @pl.when(k > 0)
                def _():
                    w_wait(c - GR, b)

                g_start(c, b)
            for b in range(GR):
                c = k * GR + b
                g_wait(c, b)
                w_start(c, b)
            return carry

        lax.fori_loop(0, ng, body, 0)
        for t in range(tail):
            c = ng * GR + t
            w_wait(c - GR, t)
            g_start(c, t)
            g_wait(c, t)
            w_start(c, t)
        for b in range(tail, GR):
            w_wait(ng * GR - GR + b, b)
        for t in range(tail):
            w_wait(ng * GR + t, t)

    return gather_k


def _make_scatter(n_nodes, n_edges, nc, ns):
    nw = nc * ns
    per_w = n_edges // nw
    iters = per_w // C
    n_chunks = n_nodes // NR
    chunks_per_tile = (n_chunks + ns - 1) // ns
    GR = 3
    ng = iters // GR
    tail = iters - ng * GR

    @functools.partial(
        pl.kernel,
        mesh=_sc_mesh(),
        compiler_params=pltpu.CompilerParams(needs_layout_passes=False),
        out_type=(
            jax.ShapeDtypeStruct((2, n_nodes, D), jnp.float32),
            jax.ShapeDtypeStruct((2, n_nodes), jnp.float32),
        ),
        scratch_types=[
            pltpu.VMEM((iters, C), jnp.int32),
            pltpu.VMEM((C * 8,), jnp.float32),
            pltpu.VMEM((C * 8,), jnp.float32),
            pltpu.VMEM((C * 8,), jnp.float32),
            pltpu.VMEM((C,), jnp.float32),
            pltpu.VMEM((C,), jnp.float32),
            pltpu.VMEM((C,), jnp.float32),
            pltpu.VMEM((C, D), jnp.float32),
            pltpu.VMEM((C, D), jnp.float32),
            pltpu.VMEM((C, D), jnp.float32),
            pltpu.VMEM_SHARED((n_nodes, D), jnp.float32),
            pltpu.VMEM_SHARED((n_nodes,), jnp.float32),
            pltpu.SemaphoreType.DMA,
            pltpu.SemaphoreType.DMA,
            pltpu.SemaphoreType.DMA,
            pltpu.SemaphoreType.DMA,
            pltpu.SemaphoreType.DMA,
            pltpu.SemaphoreType.DMA,
        ],
    )
    def scatter_k(p_hbm, tgt3_hbm, e_hbm, z2_hbm, z1_hbm,
                  part_hbm, den_hbm, idx2, ec0, ec1, ec2,
                  eb0, eb1, eb2, r0, r1, r2, acc, dacc,
                  ls0, ls1, ls2, ss0, ss1, ss2):
        e8c = [ec0, ec1, ec2]
        eb = [eb0, eb1, eb2]
        rows = [r0, r1, r2]
        lsem = [ls0, ls1, ls2]
        ssem = [ss0, ss1, ss2]
        cid = lax.axis_index("c")
        sid = lax.axis_index("s")
        wid = sid * nc + cid
        lanes = lax.iota(jnp.int32, 16)

        pltpu.sync_copy(tgt3_hbm.at[wid], idx2)

        def zero(k, carry):
            chunk = sid + k * ns

            @pl.when(chunk < n_chunks)
            def _():
                lo = chunk * NR
                pltpu.sync_copy(z2_hbm.at[pl.ds(lo, NR)],
                                acc.at[pl.ds(lo, NR)])

            return carry

        lax.fori_loop(0, chunks_per_tile, zero, 0)

        @pl.when(sid == 0)
        def _():
            pltpu.sync_copy(z1_hbm, dacc)

        plsc.subcore_barrier()

        def l_start(c, b):
            pltpu.async_copy(
                p_hbm.at[pl.ds(wid * per_w + c * C, C)], rows[b], lsem[b])
            pltpu.async_copy(
                e_hbm.at[pl.ds((wid * per_w + c * C) * 8, C * 8)], e8c[b],
                lsem[b])

        def l_wait(c, b):
            pltpu.make_async_copy(
                p_hbm.at[pl.ds(wid * per_w + c * C, C)], rows[b],
                lsem[b]).wait()
            pltpu.make_async_copy(
                e_hbm.at[pl.ds((wid * per_w + c * C) * 8, C * 8)], e8c[b],
                lsem[b]).wait()

        def s_start(c, b):
            for j in range(C // 16):
                row = (lanes + (j * 16)) * 8
                eb[b][pl.ds(j * 16, 16)] = plsc.load_gather(e8c[b], [row])
            pltpu.async_copy(rows[b], acc.at[idx2.at[c]], ssem[b],
                             add=True)
            pltpu.async_copy(eb[b], dacc.at[idx2.at[c]], ssem[b],
                             add=True)

        def s_wait(c, b):
            pltpu.make_async_copy(rows[b], acc.at[idx2.at[c]],
                                  ssem[b]).wait()
            pltpu.make_async_copy(eb[b], dacc.at[idx2.at[c]],
                                  ssem[b]).wait()

        def body(k, carry):
            for b in range(GR):
                c = k * GR + b

                @pl.when(k > 0)
                def _():
                    s_wait(c - GR, b)

                l_start(c, b)
            for b in range(GR):
                c = k * GR + b
                l_wait(c, b)
                s_start(c, b)
            return carry

        lax.fori_loop(0, ng, body, 0)
        for t in range(tail):
            c = ng * GR + t
            s_wait(c - GR, t)
            l_start(c, t)
            l_wait(c, t)
            s_start(c, t)
        for b in range(tail, GR):
            s_wait(ng * GR - GR + b, b)
        for t in range(tail):
            s_wait(ng * GR + t, t)

        plsc.subcore_barrier()

        def wout(k, carry):
            chunk = sid + k * ns

            @pl.when(chunk < n_chunks)
            def _():
                lo = chunk * NR
                pltpu.sync_copy(acc.at[pl.ds(lo, NR)],
                                part_hbm.at[cid, pl.ds(lo, NR)])

            return carry

        lax.fori_loop(0, chunks_per_tile, wout, 0)

        @pl.when(sid == 0)
        def _():
            pltpu.sync_copy(dacc, den_hbm.at[cid])

    return scatter_k


def _make_finalize(n_nodes, n_edges, nc, ns):
    nw = nc * ns
    per_w = n_edges // nw
    iters = per_w // C
    n_chunks = n_nodes // NR
    chunks_per_w = (n_chunks + nw - 1) // nw

    @functools.partial(
        pl.kernel,
        mesh=_sc_mesh(),
        compiler_params=pltpu.CompilerParams(needs_layout_passes=False),
        out_type=(
            jax.ShapeDtypeStruct((n_nodes, D), jnp.float32),
            jax.ShapeDtypeStruct((n_edges,), jnp.float32),
        ),
        scratch_types=[
            pltpu.VMEM((n_nodes,), jnp.float32),
            pltpu.VMEM((NR, D), jnp.float32),
            pltpu.VMEM((NR, D), jnp.float32),
            pltpu.VMEM((per_w * 8,), jnp.float32),
            pltpu.VMEM((per_w,), jnp.int32),
            pltpu.VMEM((per_w,), jnp.float32),
        ],
    )
    def fin_k(part_hbm, denp_hbm, e_hbm, tgt_hbm, out_hbm, att_hbm,
              den_v, r0_v, r1_v, e8_all, t_all, a_all):
        wid = lax.axis_index("s") * nc + lax.axis_index("c")
        lanes = lax.iota(jnp.int32, 16)
        zero16 = lanes * 0

        pltpu.sync_copy(denp_hbm.at[0], den_v)
        pltpu.sync_copy(denp_hbm.at[1], a_all)
        pltpu.sync_copy(e_hbm.at[pl.ds(wid * per_w * 8, per_w * 8)], e8_all)
        pltpu.sync_copy(tgt_hbm.at[pl.ds(wid * per_w, per_w)], t_all)

        def dsum(i, carry):
            sl = pl.ds(i * 16, 16)
            den_v[sl] = den_v[sl] + a_all[sl]
            return carry

        lax.fori_loop(0, n_nodes // 16, dsum, 0)

        def rows(k, carry):
            chunk = wid + k * nw

            @pl.when(chunk < n_chunks)
            def _():
                lo = chunk * NR
                pltpu.sync_copy(part_hbm.at[0, pl.ds(lo, NR)], r0_v)
                pltpu.sync_copy(part_hbm.at[1, pl.ds(lo, NR)], r1_v)
                for r in range(NR):
                    dv = plsc.load_gather(den_v, [zero16 + (lo + r)])
                    ok = dv > 0.0
                    dsafe = jnp.where(ok, dv, 1.0)
                    for cc in range(D // 16):
                        sl = pl.ds(cc * 16, 16)
                        q = (r0_v[r, sl] + r1_v[r, sl]) / dsafe
                        r0_v[r, sl] = jnp.where(ok, q, 0.0)
                pltpu.sync_copy(r0_v, out_hbm.at[pl.ds(lo, NR)])

            return carry

        lax.fori_loop(0, chunks_per_w, rows, 0)

        def att(i, carry):
            sl = pl.ds(i * 16, 16)
            dv = plsc.load_gather(den_v, [t_all[sl]])
            ev = plsc.load_gather(e8_all, [(lanes + i * 16) * 8])
            a_all[sl] = ev / dv
            return carry

        lax.fori_loop(0, per_w // 16, att, 0)
        pltpu.sync_copy(a_all, att_hbm.at[pl.ds(wid * per_w, per_w)])

    return fin_k


def _tc_edge_math(g, edge_attr, w1t, b1, w2t, b2, att):
    n_edges = g.shape[0]

    def body(g_ref, ea_ref, w1t_ref, b1_ref, w2t_ref, b2_ref, att_ref,
             p_ref, e_ref):
        h = jnp.tanh(
            jnp.dot(ea_ref[...], w1t_ref[...],
                    preferred_element_type=jnp.float32) + b1_ref[...])
        w = jnp.dot(h, w2t_ref[...],
                    preferred_element_type=jnp.float32) + b2_ref[...]
        m = g_ref[...] * w
        e_full = jnp.exp(jnp.dot(m, att_ref[...],
                                 preferred_element_type=jnp.float32))
        p_ref[...] = m * e_full
        e_ref[...] = e_full[:, 0:8]

    grid = (n_edges // TE,)
    return pl.pallas_call(
        body,
        grid=grid,
        in_specs=[
            pl.BlockSpec((TE, D), lambda i: (i, 0)),
            pl.BlockSpec((TE, 16), lambda i: (i, 0)),
            pl.BlockSpec((16, 128), lambda i: (0, 0)),
            pl.BlockSpec((1, 128), lambda i: (0, 0)),
            pl.BlockSpec((128, 128), lambda i: (0, 0)),
            pl.BlockSpec((1, 128), lambda i: (0, 0)),
            pl.BlockSpec((128, 128), lambda i: (0, 0)),
        ],
        out_specs=[
            pl.BlockSpec((TE, D), lambda i: (i, 0)),
            pl.BlockSpec((TE, 8), lambda i: (i, 0)),
        ],
        out_shape=[
            jax.ShapeDtypeStruct((n_edges, D), jnp.float32),
            jax.ShapeDtypeStruct((n_edges, 8), jnp.float32),
        ],
    )(g, edge_attr, w1t, b1, w2t, b2, att)


def kernel(x, edge_index, edge_attr, W1, b1, W2, b2, att_vec):
    n_nodes = x.shape[0]
    n_edges = edge_attr.shape[0]
    info = plsc.get_sparse_core_info()
    nc, ns = info.num_cores, info.num_subcores
    nw = nc * ns
    per_w = n_edges // nw

    src = edge_index[0].astype(jnp.int32)
    tgt = edge_index[1].astype(jnp.int32)
    tgt3 = tgt.reshape(nw, per_w // C, C)

    g = _make_gather(n_nodes, n_edges, nc, ns)(x, src)

    att_mat = jnp.tile(att_vec, (1, 128))
    p, e8 = _tc_edge_math(
        g, edge_attr, W1.T, b1.reshape(1, 128), W2.T, b2.reshape(1, 128),
        att_mat)

    e8f = e8.reshape(-1)
    z2 = jnp.zeros((n_nodes, D), jnp.float32)
    z1 = jnp.zeros((n_nodes,), jnp.float32)
    part, den_part = _make_scatter(n_nodes, n_edges, nc, ns)(
        p, tgt3, e8f, z2, z1)

    out, att = _make_finalize(n_nodes, n_edges, nc, ns)(
        part, den_part, e8f, tgt)

    return (out, att)


# speedup vs baseline: 10.6171x; 1.0234x over previous
import functools

import jax
import jax.numpy as jnp
from jax import lax
from jax.experimental import pallas as pl
from jax.experimental.pallas import tpu as pltpu
from jax.experimental.pallas import tpu_sc as plsc

D = 128
C = 80
TE = 6400
NR = 40


def _sc_mesh():
    return plsc.VectorSubcoreMesh(core_axis_name="c", subcore_axis_name="s")


def _make_gather(n_nodes, n_edges, nc, ns):
    nw = nc * ns
    per_w = n_edges // nw
    iters = per_w // C
    n_chunks = n_nodes // NR
    chunks_per_tile = (n_chunks + ns - 1) // ns
    GR = 3
    ng = iters // GR
    tail = iters - ng * GR

    @functools.partial(
        pl.kernel,
        mesh=_sc_mesh(),
        compiler_params=pltpu.CompilerParams(needs_layout_passes=False),
        out_type=jax.ShapeDtypeStruct((n_edges, D), jnp.float32),
        scratch_types=[
            pltpu.VMEM((per_w,), jnp.int32),
            pltpu.VMEM((C, D), jnp.float32),
            pltpu.VMEM((C, D), jnp.float32),
            pltpu.VMEM((C, D), jnp.float32),
            pltpu.VMEM_SHARED((n_nodes, D), jnp.float32),
            pltpu.SemaphoreType.DMA,
            pltpu.SemaphoreType.DMA,
            pltpu.SemaphoreType.DMA,
            pltpu.SemaphoreType.DMA,
            pltpu.SemaphoreType.DMA,
            pltpu.SemaphoreType.DMA,
        ],
    )
    def gather_k(x_hbm, src_hbm, g_hbm, idx_all, r0, r1, r2, xs,
                 gs0, gs1, gs2, ws0, ws1, ws2):
        rows = [r0, r1, r2]
        gsem = [gs0, gs1, gs2]
        wsem = [ws0, ws1, ws2]
        cid = lax.axis_index("c")
        sid = lax.axis_index("s")
        wid = sid * nc + cid
        pltpu.sync_copy(src_hbm.at[pl.ds(wid * per_w, per_w)], idx_all)

        def stage(k, carry):
            chunk = sid + k * ns

            @pl.when(chunk < n_chunks)
            def _():
                lo = chunk * NR
                pltpu.sync_copy(x_hbm.at[pl.ds(lo, NR)], xs.at[pl.ds(lo, NR)])

            return carry

        lax.fori_loop(0, chunks_per_tile, stage, 0)
        plsc.subcore_barrier()

        def g_start(c, b):
            pltpu.async_copy(
                xs.at[idx_all.at[pl.ds(c * C, C)]], rows[b], gsem[b])

        def g_wait(c, b):
            pltpu.make_async_copy(
                xs.at[idx_all.at[pl.ds(c * C, C)]], rows[b], gsem[b]).wait()

        def w_start(c, b):
            pltpu.async_copy(
                rows[b], g_hbm.at[pl.ds(wid * per_w + c * C, C)], wsem[b])

        def w_wait(c, b):
            pltpu.make_async_copy(
                rows[b], g_hbm.at[pl.ds(wid * per_w + c * C, C)],
                wsem[b]).wait()

        def body(k, carry):
            for b in range(GR):
                c = k * GR + b

                @pl.when(k > 0)
                def _():
                    w_wait(c - GR, b)

                g_start(c, b)
            for b in range(GR):
                c = k * GR + b
                g_wait(c, b)
                w_start(c, b)
            return carry

        lax.fori_loop(0, ng, body, 0)
        for t in range(tail):
            c = ng * GR + t
            w_wait(c - GR, t)
            g_start(c, t)
            g_wait(c, t)
            w_start(c, t)
        for b in range(tail, GR):
            w_wait(ng * GR - GR + b, b)
        for t in range(tail):
            w_wait(ng * GR + t, t)

    return gather_k


def _make_scatter(n_nodes, n_edges, nc, ns):
    nw = nc * ns
    per_w = n_edges // nw
    iters = per_w // C
    n_chunks = n_nodes // NR
    chunks_per_tile = (n_chunks + ns - 1) // ns
    GR = 3
    ng = iters // GR
    tail = iters - ng * GR

    @functools.partial(
        pl.kernel,
        mesh=_sc_mesh(),
        compiler_params=pltpu.CompilerParams(needs_layout_passes=False),
        out_type=(
            jax.ShapeDtypeStruct((2, n_nodes, D), jnp.float32),
            jax.ShapeDtypeStruct((2, n_nodes), jnp.float32),
            jax.ShapeDtypeStruct((n_edges,), jnp.float32),
        ),
        scratch_types=[
            pltpu.VMEM((iters, C), jnp.int32),
            pltpu.VMEM((1000,), jnp.float32),
            pltpu.VMEM((C * 8,), jnp.float32),
            pltpu.VMEM((C * 8,), jnp.float32),
            pltpu.VMEM((C * 8,), jnp.float32),
            pltpu.VMEM((C,), jnp.float32),
            pltpu.VMEM((C,), jnp.float32),
            pltpu.VMEM((C,), jnp.float32),
            pltpu.VMEM((C, D), jnp.float32),
            pltpu.VMEM((C, D), jnp.float32),
            pltpu.VMEM((C, D), jnp.float32),
            pltpu.VMEM_SHARED((n_nodes, D), jnp.float32),
            pltpu.VMEM_SHARED((n_nodes,), jnp.float32),
            pltpu.SemaphoreType.DMA,
            pltpu.SemaphoreType.DMA,
            pltpu.SemaphoreType.DMA,
            pltpu.SemaphoreType.DMA,
            pltpu.SemaphoreType.DMA,
            pltpu.SemaphoreType.DMA,
            pltpu.SemaphoreType.DMA,
            pltpu.SemaphoreType.DMA,
            pltpu.SemaphoreType.DMA,
        ],
    )
    def scatter_k(p_hbm, tgt3_hbm, e_hbm,
                  part_hbm, den_hbm, ecomp_hbm, idx2, zv, ec0, ec1, ec2,
                  eb0, eb1, eb2, r0, r1, r2, acc, dacc,
                  ls0, ls1, ls2, ss0, ss1, ss2, es0, es1, es2):
        e8c = [ec0, ec1, ec2]
        eb = [eb0, eb1, eb2]
        rows = [r0, r1, r2]
        lsem = [ls0, ls1, ls2]
        ssem = [ss0, ss1, ss2]
        esem = [es0, es1, es2]
        cid = lax.axis_index("c")
        sid = lax.axis_index("s")
        wid = sid * nc + cid
        lanes = lax.iota(jnp.int32, 16)

        pltpu.sync_copy(tgt3_hbm.at[wid], idx2)

        z16 = jnp.zeros((16,), jnp.float32)
        for j in range(1000 // 16 + 1):
            zv[pl.ds(min(j * 16, 1000 - 16), 16)] = z16
        for r in range(NR):
            for cc in range(D // 16):
                r0[r, pl.ds(cc * 16, 16)] = z16

        def zero(k, carry):
            chunk = sid + k * ns

            @pl.when(chunk < n_chunks)
            def _():
                lo = chunk * NR
                pltpu.sync_copy(r0.at[pl.ds(0, NR)], acc.at[pl.ds(lo, NR)])

            return carry

        lax.fori_loop(0, chunks_per_tile, zero, 0)

        @pl.when(sid == 0)
        def _():
            for j in range(n_nodes // 1000):
                pltpu.sync_copy(zv, dacc.at[pl.ds(j * 1000, 1000)])

        plsc.subcore_barrier()

        def l_start(c, b):
            pltpu.async_copy(
                p_hbm.at[pl.ds(wid * per_w + c * C, C)], rows[b], lsem[b])
            pltpu.async_copy(
                e_hbm.at[pl.ds((wid * per_w + c * C) * 8, C * 8)], e8c[b],
                lsem[b])

        def l_wait(c, b):
            pltpu.make_async_copy(
                p_hbm.at[pl.ds(wid * per_w + c * C, C)], rows[b],
                lsem[b]).wait()
            pltpu.make_async_copy(
                e_hbm.at[pl.ds((wid * per_w + c * C) * 8, C * 8)], e8c[b],
                lsem[b]).wait()

        def s_start(c, b):
            for j in range(C // 16):
                row = (lanes + (j * 16)) * 8
                eb[b][pl.ds(j * 16, 16)] = plsc.load_gather(e8c[b], [row])
            pltpu.async_copy(rows[b], acc.at[idx2.at[c]], ssem[b],
                             add=True)
            pltpu.async_copy(eb[b], dacc.at[idx2.at[c]], ssem[b],
                             add=True)
            pltpu.async_copy(
                eb[b], ecomp_hbm.at[pl.ds(wid * per_w + c * C, C)], esem[b])

        def s_wait(c, b):
            pltpu.make_async_copy(rows[b], acc.at[idx2.at[c]],
                                  ssem[b]).wait()
            pltpu.make_async_copy(eb[b], dacc.at[idx2.at[c]],
                                  ssem[b]).wait()
            pltpu.make_async_copy(
                eb[b], ecomp_hbm.at[pl.ds(wid * per_w + c * C, C)],
                esem[b]).wait()

        def body(k, carry):
            for b in range(GR):
                c = k * GR + b

                @pl.when(k > 0)
                def _():
                    s_wait(c - GR, b)

                l_start(c, b)
            for b in range(GR):
                c = k * GR + b
                l_wait(c, b)
                s_start(c, b)
            return carry

        lax.fori_loop(0, ng, body, 0)
        for t in range(tail):
            c = ng * GR + t
            s_wait(c - GR, t)
            l_start(c, t)
            l_wait(c, t)
            s_start(c, t)
        for b in range(tail, GR):
            s_wait(ng * GR - GR + b, b)
        for t in range(tail):
            s_wait(ng * GR + t, t)

        plsc.subcore_barrier()

        def wout(k, carry):
            chunk = sid + k * ns

            @pl.when(chunk < n_chunks)
            def _():
                lo = chunk * NR
                pltpu.sync_copy(acc.at[pl.ds(lo, NR)],
                                part_hbm.at[cid, pl.ds(lo, NR)])

            return carry

        lax.fori_loop(0, chunks_per_tile, wout, 0)

        @pl.when(sid == 0)
        def _():
            pltpu.sync_copy(dacc, den_hbm.at[cid])

    return scatter_k


def _make_finalize(n_nodes, n_edges, nc, ns):
    nw = nc * ns
    per_w = n_edges // nw
    iters = per_w // C
    n_chunks = n_nodes // NR
    chunks_per_w = (n_chunks + nw - 1) // nw

    @functools.partial(
        pl.kernel,
        mesh=_sc_mesh(),
        compiler_params=pltpu.CompilerParams(needs_layout_passes=False),
        out_type=(
            jax.ShapeDtypeStruct((n_nodes, D), jnp.float32),
            jax.ShapeDtypeStruct((n_edges,), jnp.float32),
        ),
        scratch_types=[
            pltpu.VMEM((n_nodes,), jnp.float32),
            pltpu.VMEM((NR, D), jnp.float32),
            pltpu.VMEM((NR, D), jnp.float32),
            pltpu.VMEM((per_w,), jnp.float32),
            pltpu.VMEM((per_w,), jnp.int32),
            pltpu.VMEM((per_w,), jnp.float32),
        ],
    )
    def fin_k(part_hbm, denp_hbm, e_hbm, tgt_hbm, out_hbm, att_hbm,
              den_v, r0_v, r1_v, e_all, t_all, a_all):
        wid = lax.axis_index("s") * nc + lax.axis_index("c")
        lanes = lax.iota(jnp.int32, 16)
        zero16 = lanes * 0

        pltpu.sync_copy(denp_hbm.at[0], den_v)
        pltpu.sync_copy(denp_hbm.at[1], a_all)
        pltpu.sync_copy(e_hbm.at[pl.ds(wid * per_w, per_w)], e_all)
        pltpu.sync_copy(tgt_hbm.at[pl.ds(wid * per_w, per_w)], t_all)

        def dsum(i, carry):
            sl = pl.ds(i * 16, 16)
            den_v[sl] = den_v[sl] + a_all[sl]
            return carry

        lax.fori_loop(0, n_nodes // 16, dsum, 0)

        def rows(k, carry):
            chunk = wid + k * nw

            @pl.when(chunk < n_chunks)
            def _():
                lo = chunk * NR
                pltpu.sync_copy(part_hbm.at[0, pl.ds(lo, NR)], r0_v)
                pltpu.sync_copy(part_hbm.at[1, pl.ds(lo, NR)], r1_v)
                for r in range(NR):
                    dv = plsc.load_gather(den_v, [zero16 + (lo + r)])
                    ok = dv > 0.0
                    dsafe = jnp.where(ok, dv, 1.0)
                    for cc in range(D // 16):
                        sl = pl.ds(cc * 16, 16)
                        q = (r0_v[r, sl] + r1_v[r, sl]) / dsafe
                        r0_v[r, sl] = jnp.where(ok, q, 0.0)
                pltpu.sync_copy(r0_v, out_hbm.at[pl.ds(lo, NR)])

            return carry

        lax.fori_loop(0, chunks_per_w, rows, 0)

        def att(i, carry):
            sl = pl.ds(i * 16, 16)
            dv = plsc.load_gather(den_v, [t_all[sl]])
            a_all[sl] = e_all[sl] / dv
            return carry

        lax.fori_loop(0, per_w // 16, att, 0)
        pltpu.sync_copy(a_all, att_hbm.at[pl.ds(wid * per_w, per_w)])

    return fin_k


def _tc_edge_math(g, edge_attr, w1t, b1, w2t, b2, att):
    n_edges = g.shape[0]

    def body(g_ref, ea_ref, w1t_ref, b1_ref, w2t_ref, b2_ref, att_ref,
             p_ref, e_ref):
        h = jnp.tanh(
            jnp.dot(ea_ref[...], w1t_ref[...],
                    preferred_element_type=jnp.float32) + b1_ref[...])
        w = jnp.dot(h, w2t_ref[...],
                    preferred_element_type=jnp.float32) + b2_ref[...]
        m = g_ref[...] * w
        e_full = jnp.exp(jnp.dot(m, att_ref[...],
                                 preferred_element_type=jnp.float32))
        p_ref[...] = m * e_full
        e_ref[...] = e_full[:, 0:8]

    grid = (n_edges // TE,)
    return pl.pallas_call(
        body,
        grid=grid,
        in_specs=[
            pl.BlockSpec((TE, D), lambda i: (i, 0)),
            pl.BlockSpec((TE, 16), lambda i: (i, 0)),
            pl.BlockSpec((16, 128), lambda i: (0, 0)),
            pl.BlockSpec((1, 128), lambda i: (0, 0)),
            pl.BlockSpec((128, 128), lambda i: (0, 0)),
            pl.BlockSpec((1, 128), lambda i: (0, 0)),
            pl.BlockSpec((128, 128), lambda i: (0, 0)),
        ],
        out_specs=[
            pl.BlockSpec((TE, D), lambda i: (i, 0)),
            pl.BlockSpec((TE, 8), lambda i: (i, 0)),
        ],
        out_shape=[
            jax.ShapeDtypeStruct((n_edges, D), jnp.float32),
            jax.ShapeDtypeStruct((n_edges, 8), jnp.float32),
        ],
    )(g, edge_attr, w1t, b1, w2t, b2, att)


def kernel(x, edge_index, edge_attr, W1, b1, W2, b2, att_vec):
    n_nodes = x.shape[0]
    n_edges = edge_attr.shape[0]
    info = plsc.get_sparse_core_info()
    nc, ns = info.num_cores, info.num_subcores
    nw = nc * ns
    per_w = n_edges // nw

    src = edge_index[0].astype(jnp.int32)
    tgt = edge_index[1].astype(jnp.int32)
    tgt3 = tgt.reshape(nw, per_w // C, C)

    g = _make_gather(n_nodes, n_edges, nc, ns)(x, src)

    att_mat = jnp.tile(att_vec, (1, 128))
    p, e8 = _tc_edge_math(
        g, edge_attr, W1.T, b1.reshape(1, 128), W2.T, b2.reshape(1, 128),
        att_mat)

    e8f = e8.reshape(-1)
    part, den_part, ecomp = _make_scatter(n_nodes, n_edges, nc, ns)(
        p, tgt3, e8f)

    out, att = _make_finalize(n_nodes, n_edges, nc, ns)(
        part, den_part, ecomp, tgt)

    return (out, att)
